# baseline (device time: 14224 ns/iter reference)
import jax
import jax.numpy as jnp
from jax import lax
from jax.experimental import pallas as pl
from jax.experimental.pallas import tpu as pltpu

B, SQ, SKV, H, D = 8, 1, 512, 8, 64


def kernel(Q, K, V):
    scale = D ** -0.5
    Kt = jnp.transpose(K, (0, 2, 3, 1))
    Vt = jnp.transpose(V, (0, 2, 3, 1))

    def body(q_ref, k_ref, v_ref, out_ref, kbuf, vbuf,
             send_buf, recv_buf, obuf, sems, send_sems, recv_sems, out_sems):
        my_x = lax.axis_index("x")
        my_y = lax.axis_index("y")
        my_z = lax.axis_index("z")
        nbr = (my_x, 1 - my_y, my_z)

        barrier_sem = pltpu.get_barrier_semaphore()
        pl.semaphore_signal(
            barrier_sem, inc=1, device_id=nbr, device_id_type=pl.DeviceIdType.MESH
        )

        HH = H // 2
        copies = []
        for b in range(B):
            cs = []
            for i in range(2):
                hs = slice(i * HH, (i + 1) * HH)
                kc = pltpu.make_async_copy(
                    k_ref.at[b, hs], kbuf.at[b, hs], sems.at[2 * b + i]
                )
                vc = pltpu.make_async_copy(
                    v_ref.at[b, hs], vbuf.at[b, hs], sems.at[2 * B + 2 * b + i]
                )
                kc.start()
                vc.start()
                cs.append((kc, vc))
            copies.append(cs)

        rdmas = []
        for b in range(B):
            (kc0, vc0), (kc1, vc1) = copies[b]
            q_b = q_ref[b, 0]
            kc0.wait()
            kc1.wait()
            s = lax.dot_general(
                q_b, kbuf[b],
                (((1,), (1,)), ((0,), (0,))),
                preferred_element_type=jnp.float32,
            ) * scale
            m = jnp.max(s, axis=-1, keepdims=True)
            p = jnp.exp(s - m)
            l = jnp.sum(p, axis=-1, keepdims=True)
            vc0.wait()
            vc1.wait()
            o = lax.dot_general(
                p, vbuf[b],
                (((1,), (2,)), ((0,), (0,))),
                preferred_element_type=jnp.float32,
            )
            send_buf[b, :, 0:D] = o
            send_buf[b, :, D:2 * D] = jnp.broadcast_to(m, (H, D))
            send_buf[b, :, 2 * D:3 * D] = jnp.broadcast_to(l, (H, D))
            if b == 0:
                pl.semaphore_wait(barrier_sem, 1)
            rdma = pltpu.make_async_remote_copy(
                src_ref=send_buf.at[b],
                dst_ref=recv_buf.at[b],
                send_sem=send_sems.at[b],
                recv_sem=recv_sems.at[b],
                device_id=nbr,
                device_id_type=pl.DeviceIdType.MESH,
            )
            rdma.start()
            rdmas.append(rdma)

        out_copies = []
        for b in range(B):
            rdmas[b].wait_recv()
            o1 = send_buf[b, :, 0:D]
            m1 = send_buf[b, :, D:2 * D]
            l1 = send_buf[b, :, 2 * D:3 * D]
            o2 = recv_buf[b, :, 0:D]
            m2 = recv_buf[b, :, D:2 * D]
            l2 = recv_buf[b, :, 2 * D:3 * D]
            mn = jnp.maximum(m1, m2)
            a1 = jnp.exp(m1 - mn)
            a2 = jnp.exp(m2 - mn)
            obuf[b, 0, :, :] = (a1 * o1 + a2 * o2) / (a1 * l1 + a2 * l2)
            oc = pltpu.make_async_copy(obuf.at[b], out_ref.at[b], out_sems.at[b])
            oc.start()
            out_copies.append(oc)
        for b in range(B):
            rdmas[b].wait_send()
            out_copies[b].wait()

    return pl.pallas_call(
        body,
        out_shape=jax.ShapeDtypeStruct((B, SQ, H, D), jnp.float32),
        in_specs=[
            pl.BlockSpec(memory_space=pltpu.MemorySpace.VMEM),
            pl.BlockSpec(memory_space=pl.ANY),
            pl.BlockSpec(memory_space=pl.ANY),
        ],
        out_specs=pl.BlockSpec(memory_space=pl.ANY),
        scratch_shapes=[
            pltpu.VMEM((B, H, D, SKV), jnp.float32),
            pltpu.VMEM((B, H, D, SKV), jnp.float32),
            pltpu.VMEM((B, H, 3 * D), jnp.float32),
            pltpu.VMEM((B, H, 3 * D), jnp.float32),
            pltpu.VMEM((B, SQ, H, D), jnp.float32),
            pltpu.SemaphoreType.DMA((4 * B,)),
            pltpu.SemaphoreType.DMA((B,)),
            pltpu.SemaphoreType.DMA((B,)),
            pltpu.SemaphoreType.DMA((B,)),
        ],
        compiler_params=pltpu.CompilerParams(
            collective_id=0,
            vmem_limit_bytes=96 * 1024 * 1024,
        ),
    )(Q, Kt, Vt)


# device time: 13955 ns/iter; 1.0193x vs baseline; 1.0193x over previous
import jax
import jax.numpy as jnp
from jax import lax
from jax.experimental import pallas as pl
from jax.experimental.pallas import tpu as pltpu

B, SQ, SKV, H, D = 8, 1, 512, 8, 64


def kernel(Q, K, V):
    scale = D ** -0.5
    Kt = jnp.transpose(K, (0, 2, 3, 1))
    Vt = jnp.transpose(V, (0, 2, 3, 1))

    def body(q_ref, k_ref, v_ref, out_ref, kbuf, vbuf,
             send_buf, recv_buf, obuf, sems, send_sems, recv_sems, out_sems):
        my_x = lax.axis_index("x")
        my_y = lax.axis_index("y")
        my_z = lax.axis_index("z")
        nbr = (my_x, 1 - my_y, my_z)

        barrier_sem = pltpu.get_barrier_semaphore()
        pl.semaphore_signal(
            barrier_sem, inc=1, device_id=nbr, device_id_type=pl.DeviceIdType.MESH
        )

        copies = []
        for b in range(B):
            kc = pltpu.make_async_copy(k_ref.at[b], kbuf.at[b], sems.at[b])
            vc = pltpu.make_async_copy(v_ref.at[b], vbuf.at[b], sems.at[B + b])
            kc.start()
            vc.start()
            copies.append((kc, vc))

        rdmas = []
        for b in range(B):
            kc, vc = copies[b]
            q_b = q_ref[b, 0]
            kc.wait()
            s = lax.dot_general(
                q_b, kbuf[b],
                (((1,), (1,)), ((0,), (0,))),
                preferred_element_type=jnp.float32,
            ) * scale
            m = jnp.max(s, axis=-1, keepdims=True)
            p = jnp.exp(s - m)
            l = jnp.sum(p, axis=-1, keepdims=True)
            vc.wait()
            o = lax.dot_general(
                p, vbuf[b],
                (((1,), (2,)), ((0,), (0,))),
                preferred_element_type=jnp.float32,
            )
            send_buf[b, :, 0:D] = o
            send_buf[b, :, D:2 * D] = jnp.broadcast_to(m, (H, D))
            send_buf[b, :, 2 * D:3 * D] = jnp.broadcast_to(l, (H, D))
            if b == 0:
                pl.semaphore_wait(barrier_sem, 1)
            rdma = pltpu.make_async_remote_copy(
                src_ref=send_buf.at[b],
                dst_ref=recv_buf.at[b],
                send_sem=send_sems.at[b],
                recv_sem=recv_sems.at[b],
                device_id=nbr,
                device_id_type=pl.DeviceIdType.MESH,
            )
            rdma.start()
            rdmas.append(rdma)

        out_copies = []
        for b in range(B):
            rdmas[b].wait_recv()
            o1 = send_buf[b, :, 0:D]
            m1 = send_buf[b, :, D:2 * D]
            l1 = send_buf[b, :, 2 * D:3 * D]
            o2 = recv_buf[b, :, 0:D]
            m2 = recv_buf[b, :, D:2 * D]
            l2 = recv_buf[b, :, 2 * D:3 * D]
            mn = jnp.maximum(m1, m2)
            a1 = jnp.exp(m1 - mn)
            a2 = jnp.exp(m2 - mn)
            obuf[b, 0, :, :] = (a1 * o1 + a2 * o2) / (a1 * l1 + a2 * l2)
            oc = pltpu.make_async_copy(obuf.at[b], out_ref.at[b], out_sems.at[b])
            oc.start()
            out_copies.append(oc)
        for b in range(B):
            rdmas[b].wait_send()
            out_copies[b].wait()

    return pl.pallas_call(
        body,
        out_shape=jax.ShapeDtypeStruct((B, SQ, H, D), jnp.float32),
        in_specs=[
            pl.BlockSpec(memory_space=pltpu.MemorySpace.VMEM),
            pl.BlockSpec(memory_space=pl.ANY),
            pl.BlockSpec(memory_space=pl.ANY),
        ],
        out_specs=pl.BlockSpec(memory_space=pl.ANY),
        scratch_shapes=[
            pltpu.VMEM((B, H, D, SKV), jnp.float32),
            pltpu.VMEM((B, H, D, SKV), jnp.float32),
            pltpu.VMEM((B, H, 3 * D), jnp.float32),
            pltpu.VMEM((B, H, 3 * D), jnp.float32),
            pltpu.VMEM((B, SQ, H, D), jnp.float32),
            pltpu.SemaphoreType.DMA((2 * B,)),
            pltpu.SemaphoreType.DMA((B,)),
            pltpu.SemaphoreType.DMA((B,)),
            pltpu.SemaphoreType.DMA((B,)),
        ],
        compiler_params=pltpu.CompilerParams(
            collective_id=0,
            vmem_limit_bytes=96 * 1024 * 1024,
        ),
    )(Q, Kt, Vt)


# device time: 13488 ns/iter; 1.0546x vs baseline; 1.0346x over previous
import jax
import jax.numpy as jnp
from jax import lax
from jax.experimental import pallas as pl
from jax.experimental.pallas import tpu as pltpu

B, SQ, SKV, H, D = 8, 1, 512, 8, 64
NB = B // 2


def kernel(Q, K, V):
    scale = D ** -0.5
    Kt = jnp.transpose(K, (0, 2, 3, 1))
    Vt = jnp.transpose(V, (0, 2, 3, 1))

    def body(q_ref, k_ref, v_ref, out_ref, kbuf, vbuf,
             send_buf, recv_buf, obuf,
             sems, ysend_sems, yrecv_sems, xsend_sems, xrecv_sems, out_sems):
        my_x = lax.axis_index("x")
        my_y = lax.axis_index("y")
        my_z = lax.axis_index("z")
        nbr_y = (my_x, 1 - my_y, my_z)
        nbr_x = (1 - my_x, my_y, my_z)
        b0 = my_x * NB

        barrier_sem = pltpu.get_barrier_semaphore()
        for nbr in (nbr_y, nbr_x):
            pl.semaphore_signal(
                barrier_sem, inc=1, device_id=nbr,
                device_id_type=pl.DeviceIdType.MESH,
            )

        copies = []
        for i in range(NB):
            kc = pltpu.make_async_copy(
                k_ref.at[b0 + i], kbuf.at[i], sems.at[i]
            )
            vc = pltpu.make_async_copy(
                v_ref.at[b0 + i], vbuf.at[i], sems.at[NB + i]
            )
            kc.start()
            vc.start()
            copies.append((kc, vc))

        y_rdmas = []
        for i in range(NB):
            kc, vc = copies[i]
            q_b = q_ref[pl.ds(b0 + i, 1)].reshape(H, D)
            kc.wait()
            s = lax.dot_general(
                q_b, kbuf[i],
                (((1,), (1,)), ((0,), (0,))),
                preferred_element_type=jnp.float32,
            ) * scale
            m = jnp.max(s, axis=-1, keepdims=True)
            p = jnp.exp(s - m)
            l = jnp.sum(p, axis=-1, keepdims=True)
            vc.wait()
            o = lax.dot_general(
                p, vbuf[i],
                (((1,), (2,)), ((0,), (0,))),
                preferred_element_type=jnp.float32,
            )
            send_buf[i, :, 0:D] = o
            send_buf[i, :, D:2 * D] = jnp.broadcast_to(m, (H, D))
            send_buf[i, :, 2 * D:3 * D] = jnp.broadcast_to(l, (H, D))
            if i == 0:
                pl.semaphore_wait(barrier_sem, 2)
            rdma = pltpu.make_async_remote_copy(
                src_ref=send_buf.at[i],
                dst_ref=recv_buf.at[i],
                send_sem=ysend_sems.at[i],
                recv_sem=yrecv_sems.at[i],
                device_id=nbr_y,
                device_id_type=pl.DeviceIdType.MESH,
            )
            rdma.start()
            y_rdmas.append(rdma)

        x_rdmas = []
        out_copies = []
        for i in range(NB):
            y_rdmas[i].wait_recv()
            o1 = send_buf[i, :, 0:D]
            m1 = send_buf[i, :, D:2 * D]
            l1 = send_buf[i, :, 2 * D:3 * D]
            o2 = recv_buf[i, :, 0:D]
            m2 = recv_buf[i, :, D:2 * D]
            l2 = recv_buf[i, :, 2 * D:3 * D]
            mn = jnp.maximum(m1, m2)
            a1 = jnp.exp(m1 - mn)
            a2 = jnp.exp(m2 - mn)
            comb = (a1 * o1 + a2 * o2) / (a1 * l1 + a2 * l2)
            obuf[pl.ds(b0 + i, 1)] = comb[None]
            xr = pltpu.make_async_remote_copy(
                src_ref=obuf.at[b0 + i],
                dst_ref=obuf.at[b0 + i],
                send_sem=xsend_sems.at[i],
                recv_sem=xrecv_sems.at[i],
                device_id=nbr_x,
                device_id_type=pl.DeviceIdType.MESH,
            )
            xr.start()
            x_rdmas.append(xr)
            oc = pltpu.make_async_copy(
                obuf.at[b0 + i], out_ref.at[b0 + i, 0], out_sems.at[i]
            )
            oc.start()
            out_copies.append(oc)

        ob0 = (1 - my_x) * NB
        for i in range(NB):
            x_rdmas[i].wait_recv()
            oc = pltpu.make_async_copy(
                obuf.at[ob0 + i], out_ref.at[ob0 + i, 0], out_sems.at[NB + i]
            )
            oc.start()
            out_copies.append(oc)

        for i in range(NB):
            y_rdmas[i].wait_send()
            x_rdmas[i].wait_send()
        for oc in out_copies:
            oc.wait()

    return pl.pallas_call(
        body,
        out_shape=jax.ShapeDtypeStruct((B, SQ, H, D), jnp.float32),
        in_specs=[
            pl.BlockSpec(memory_space=pltpu.MemorySpace.VMEM),
            pl.BlockSpec(memory_space=pl.ANY),
            pl.BlockSpec(memory_space=pl.ANY),
        ],
        out_specs=pl.BlockSpec(memory_space=pl.ANY),
        scratch_shapes=[
            pltpu.VMEM((NB, H, D, SKV), jnp.float32),
            pltpu.VMEM((NB, H, D, SKV), jnp.float32),
            pltpu.VMEM((NB, H, 3 * D), jnp.float32),
            pltpu.VMEM((NB, H, 3 * D), jnp.float32),
            pltpu.VMEM((B, H, D), jnp.float32),
            pltpu.SemaphoreType.DMA((2 * NB,)),
            pltpu.SemaphoreType.DMA((NB,)),
            pltpu.SemaphoreType.DMA((NB,)),
            pltpu.SemaphoreType.DMA((NB,)),
            pltpu.SemaphoreType.DMA((NB,)),
            pltpu.SemaphoreType.DMA((2 * NB,)),
        ],
        compiler_params=pltpu.CompilerParams(
            collective_id=0,
            vmem_limit_bytes=96 * 1024 * 1024,
        ),
    )(Q, Kt, Vt)


# device time: 12402 ns/iter; 1.1469x vs baseline; 1.0876x over previous
import jax
import jax.numpy as jnp
from jax import lax
from jax.experimental import pallas as pl
from jax.experimental.pallas import tpu as pltpu

B, SQ, SKV, H, D = 8, 1, 512, 8, 64
NB = B // 2


def kernel(Q, K, V):
    scale = D ** -0.5
    Kt = jnp.transpose(K, (0, 2, 3, 1))
    Vt = jnp.transpose(V, (0, 2, 3, 1))

    def body(q_ref, k_ref, v_ref, out_ref, kbuf, vbuf,
             send_buf, ypart, xpart, dpart, obuf,
             sems, ysend_sems, yrecv_sems, xsend_sems, xrecv_sems,
             dsend_sems, drecv_sems, out_sems):
        my_x = lax.axis_index("x")
        my_y = lax.axis_index("y")
        my_z = lax.axis_index("z")
        nbr_y = (my_x, 1 - my_y, my_z)
        nbr_x = (1 - my_x, my_y, my_z)
        nbr_d = (1 - my_x, 1 - my_y, my_z)
        b0 = my_x * NB

        barrier_sem = pltpu.get_barrier_semaphore()
        for nbr in (nbr_y, nbr_x, nbr_d):
            pl.semaphore_signal(
                barrier_sem, inc=1, device_id=nbr,
                device_id_type=pl.DeviceIdType.MESH,
            )

        copies = []
        for i in range(NB):
            kc = pltpu.make_async_copy(
                k_ref.at[b0 + i], kbuf.at[i], sems.at[i]
            )
            vc = pltpu.make_async_copy(
                v_ref.at[b0 + i], vbuf.at[i], sems.at[NB + i]
            )
            kc.start()
            vc.start()
            copies.append((kc, vc))

        rdmas = []
        for i in range(NB):
            kc, vc = copies[i]
            q_b = q_ref[pl.ds(b0 + i, 1)].reshape(H, D)
            kc.wait()
            s = lax.dot_general(
                q_b, kbuf[i],
                (((1,), (1,)), ((0,), (0,))),
                preferred_element_type=jnp.float32,
            ) * scale
            m = jnp.max(s, axis=-1, keepdims=True)
            p = jnp.exp(s - m)
            l = jnp.sum(p, axis=-1, keepdims=True)
            vc.wait()
            o = lax.dot_general(
                p, vbuf[i],
                (((1,), (2,)), ((0,), (0,))),
                preferred_element_type=jnp.float32,
            )
            send_buf[i, :, 0:D] = o
            send_buf[i, :, D:2 * D] = jnp.broadcast_to(m, (H, D))
            send_buf[i, :, 2 * D:3 * D] = jnp.broadcast_to(l, (H, D))
            if i == 0:
                pl.semaphore_wait(barrier_sem, 3)
            for dst, ss, rs, nbr in (
                (ypart, ysend_sems, yrecv_sems, nbr_y),
                (xpart, xsend_sems, xrecv_sems, nbr_x),
                (dpart, dsend_sems, drecv_sems, nbr_d),
            ):
                rdma = pltpu.make_async_remote_copy(
                    src_ref=send_buf.at[i],
                    dst_ref=dst.at[i],
                    send_sem=ss.at[i],
                    recv_sem=rs.at[i],
                    device_id=nbr,
                    device_id_type=pl.DeviceIdType.MESH,
                )
                rdma.start()
                rdmas.append(rdma)

        def combine(mine, other):
            o1, m1, l1 = (mine[:, 0:D], mine[:, D:2 * D], mine[:, 2 * D:3 * D])
            o2, m2, l2 = (other[:, 0:D], other[:, D:2 * D], other[:, 2 * D:3 * D])
            mn = jnp.maximum(m1, m2)
            a1 = jnp.exp(m1 - mn)
            a2 = jnp.exp(m2 - mn)
            return (a1 * o1 + a2 * o2) / (a1 * l1 + a2 * l2)

        out_copies = []
        for i in range(NB):
            rdmas[3 * i].wait_recv()
            obuf[pl.ds(b0 + i, 1)] = combine(send_buf[i], ypart[i])[None]
            oc = pltpu.make_async_copy(
                obuf.at[b0 + i], out_ref.at[b0 + i, 0], out_sems.at[i]
            )
            oc.start()
            out_copies.append(oc)

        ob0 = (1 - my_x) * NB
        for i in range(NB):
            rdmas[3 * i + 1].wait_recv()
            rdmas[3 * i + 2].wait_recv()
            obuf[pl.ds(ob0 + i, 1)] = combine(xpart[i], dpart[i])[None]
            oc = pltpu.make_async_copy(
                obuf.at[ob0 + i], out_ref.at[ob0 + i, 0], out_sems.at[NB + i]
            )
            oc.start()
            out_copies.append(oc)

        for r in rdmas:
            r.wait_send()
        for oc in out_copies:
            oc.wait()

    return pl.pallas_call(
        body,
        out_shape=jax.ShapeDtypeStruct((B, SQ, H, D), jnp.float32),
        in_specs=[
            pl.BlockSpec(memory_space=pltpu.MemorySpace.VMEM),
            pl.BlockSpec(memory_space=pl.ANY),
            pl.BlockSpec(memory_space=pl.ANY),
        ],
        out_specs=pl.BlockSpec(memory_space=pl.ANY),
        scratch_shapes=[
            pltpu.VMEM((NB, H, D, SKV), jnp.float32),
            pltpu.VMEM((NB, H, D, SKV), jnp.float32),
            pltpu.VMEM((NB, H, 3 * D), jnp.float32),
            pltpu.VMEM((NB, H, 3 * D), jnp.float32),
            pltpu.VMEM((NB, H, 3 * D), jnp.float32),
            pltpu.VMEM((NB, H, 3 * D), jnp.float32),
            pltpu.VMEM((B, H, D), jnp.float32),
            pltpu.SemaphoreType.DMA((2 * NB,)),
            pltpu.SemaphoreType.DMA((NB,)),
            pltpu.SemaphoreType.DMA((NB,)),
            pltpu.SemaphoreType.DMA((NB,)),
            pltpu.SemaphoreType.DMA((NB,)),
            pltpu.SemaphoreType.DMA((NB,)),
            pltpu.SemaphoreType.DMA((NB,)),
            pltpu.SemaphoreType.DMA((2 * NB,)),
        ],
        compiler_params=pltpu.CompilerParams(
            collective_id=0,
            vmem_limit_bytes=96 * 1024 * 1024,
        ),
    )(Q, Kt, Vt)
